# baseline (speedup 1.0000x reference)
import functools

import jax
import jax.numpy as jnp
from jax.experimental import pallas as pl

_P = 0.5
_X = 0.5


@functools.lru_cache(maxsize=None)
def _mask_int8(bsz: int, rows: int, cols: int):
    key = jax.random.key(42)
    k_perm, k_sel = jax.random.split(key)
    n_sel = int(bsz * _X)
    indices = jax.random.permutation(k_perm, bsz)[:n_sel]
    select = jax.random.uniform(k_sel, (n_sel, rows, cols), dtype=jnp.float32)
    keep_sel = (select > _P)
    full = jnp.ones((bsz, rows, cols), dtype=jnp.bool_).at[indices].set(keep_sel)
    return jax.device_put(full.reshape(bsz, rows * cols).astype(jnp.int8))


def _mul_kernel(x_ref, m_ref, o_ref):
    o_ref[...] = x_ref[...] * m_ref[...].astype(x_ref.dtype)


def kernel(tensor):
    bsz, rows, cols = tensor.shape
    width = rows * cols
    mask = _mask_int8(bsz, rows, cols)
    x2 = tensor.reshape(bsz, width)

    block_b = 64
    while bsz % block_b:
        block_b //= 2
    grid = (bsz // block_b,)

    out = pl.pallas_call(
        _mul_kernel,
        grid=grid,
        in_specs=[
            pl.BlockSpec((block_b, width), lambda i: (i, 0)),
            pl.BlockSpec((block_b, width), lambda i: (i, 0)),
        ],
        out_specs=pl.BlockSpec((block_b, width), lambda i: (i, 0)),
        out_shape=jax.ShapeDtypeStruct((bsz, width), tensor.dtype),
    )(x2, mask)
    return out.reshape(bsz, rows, cols)
